# Initial kernel scaffold; baseline (speedup 1.0000x reference)
import jax
import jax.numpy as jnp
from jax.experimental import pallas as pl


def kernel(input, Wr, W, b):
    raise NotImplementedError("write your pallas kernel here")


# speedup vs baseline: 3.2749x; 3.2749x over previous
import functools

import jax
import jax.numpy as jnp
from jax.experimental import pallas as pl
from jax.experimental.pallas import tpu as pltpu

N = 4096
D_IN = 1024
D_OUT = 1024
E = 16
TOP_K = 2


def _router_body(x_ref, wr_ref, coeff_ref, loss_ref):
    x = x_ref[...]
    logits = jnp.dot(x, wr_ref[...], preferred_element_type=jnp.float32)
    m = jnp.max(logits, axis=1, keepdims=True)
    ex = jnp.exp(logits - m)
    probs = ex / jnp.sum(ex, axis=1, keepdims=True)

    iota = jax.lax.broadcasted_iota(jnp.int32, probs.shape, 1)
    p1 = jnp.max(probs, axis=1, keepdims=True)
    i1 = jnp.min(jnp.where(probs == p1, iota, E), axis=1, keepdims=True)
    oh1 = (iota == i1)
    masked = jnp.where(oh1, -jnp.inf, probs)
    p2 = jnp.max(masked, axis=1, keepdims=True)
    i2 = jnp.min(jnp.where(masked == p2, iota, E), axis=1, keepdims=True)
    oh2 = (iota == i2)

    wsum = p1 + p2
    w1 = p1 / wsum
    w2 = p2 / wsum
    coeff = jnp.where(oh1, w1, 0.0) + jnp.where(oh2, w2, 0.0)
    coeff_ref[...] = coeff

    counts = jnp.sum(oh1.astype(jnp.float32) + oh2.astype(jnp.float32), axis=0)
    imp = jnp.sum(probs, axis=0) / N
    loss_ref[...] = (E * jnp.sum(imp * counts) / (N * TOP_K)).reshape(1, 1)


def _mixture_body(x_ref, coeff_ref, w_ref, b_ref, out_ref):
    e = pl.program_id(1)
    coeff = coeff_ref[...]
    lane = jax.lax.broadcasted_iota(jnp.int32, coeff.shape, 1)
    c = jnp.sum(jnp.where(lane == e, coeff, 0.0), axis=1, keepdims=True)

    @pl.when(e == 0)
    def _init():
        out_ref[...] = jnp.dot(coeff_ref[...], b_ref[...],
                               preferred_element_type=jnp.float32)

    xs = x_ref[...] * c
    out_ref[...] += jnp.dot(xs, w_ref[0], preferred_element_type=jnp.float32)


def kernel(input, Wr, W, b):
    x = input
    coeff, loss = pl.pallas_call(
        _router_body,
        out_shape=(
            jax.ShapeDtypeStruct((N, E), jnp.float32),
            jax.ShapeDtypeStruct((1, 1), jnp.float32),
        ),
    )(x, Wr)

    TM = 1024
    out = pl.pallas_call(
        _mixture_body,
        grid=(N // TM, E),
        in_specs=[
            pl.BlockSpec((TM, D_IN), lambda i, j: (i, 0)),
            pl.BlockSpec((TM, E), lambda i, j: (i, 0)),
            pl.BlockSpec((1, D_IN, D_OUT), lambda i, j: (j, 0, 0)),
            pl.BlockSpec((E, D_OUT), lambda i, j: (0, 0)),
        ],
        out_specs=pl.BlockSpec((TM, D_OUT), lambda i, j: (i, 0)),
        out_shape=jax.ShapeDtypeStruct((N, D_OUT), jnp.float32),
    )(x, coeff, W, b)

    return (out, loss[0, 0])
